# baseline (speedup 1.0000x reference)
import functools
import jax
import jax.numpy as jnp
from jax import lax
from jax.experimental import pallas as pl
from jax.experimental.pallas import tpu as pltpu
from jax.experimental.pallas import tpu_sc as plsc

_NC = 2
_NS = 16
_NW = _NC * _NS
_LANES = 16
_CH = 16384


@functools.lru_cache(maxsize=None)
def _sc_posadd(batch, seq, dim):
    total = seq * dim
    seg = total // _NW
    n_chunks = seg // _CH

    mesh = plsc.VectorSubcoreMesh(core_axis_name="c", subcore_axis_name="s")

    @functools.partial(
        pl.kernel,
        out_type=jax.ShapeDtypeStruct((batch * seq * dim,), jnp.float32),
        mesh=mesh,
        scratch_types=[
            pltpu.VMEM((_CH,), jnp.float32),
            pltpu.VMEM((_CH,), jnp.float32),
        ],
    )
    def run(x_hbm, pos_hbm, out_hbm, xbuf, pbuf):
        wid = lax.axis_index("s") * _NC + lax.axis_index("c")
        seg_base = wid * seg

        def chunk_body(g, carry):
            pbase = seg_base + g * _CH
            pltpu.sync_copy(pos_hbm.at[pl.ds(pbase, _CH)], pbuf)
            for b in range(batch):
                xbase = b * total + pbase
                pltpu.sync_copy(x_hbm.at[pl.ds(xbase, _CH)], xbuf)

                def add_body(k, c):
                    o = k * _LANES
                    xbuf[pl.ds(o, _LANES)] = (
                        xbuf[pl.ds(o, _LANES)] + pbuf[pl.ds(o, _LANES)]
                    )
                    return c

                lax.fori_loop(0, _CH // _LANES, add_body, 0)
                pltpu.sync_copy(xbuf, out_hbm.at[pl.ds(xbase, _CH)])
            return carry

        lax.fori_loop(0, n_chunks, chunk_body, 0)

    return run


def kernel(x, pos_table):
    batch, seq, dim = x.shape
    run = _sc_posadd(batch, seq, dim)
    out = run(x.reshape(-1), pos_table[:seq].reshape(-1))
    return out.reshape(batch, seq, dim)


# speedup vs baseline: 1.7357x; 1.7357x over previous
import functools
import jax
import jax.numpy as jnp
from jax import lax
from jax.experimental import pallas as pl
from jax.experimental.pallas import tpu as pltpu
from jax.experimental.pallas import tpu_sc as plsc

_NC = 2
_NS = 16
_NW = _NC * _NS
_LANES = 16
_CH = 8192


@functools.lru_cache(maxsize=None)
def _sc_posadd(batch, seq, dim):
    total = seq * dim
    seg = total // _NW
    ngrp = seg // _CH
    assert total % _NW == 0 and seg % _CH == 0 and ngrp >= 4

    mesh = plsc.VectorSubcoreMesh(core_axis_name="c", subcore_axis_name="s")

    @functools.partial(
        pl.kernel,
        out_type=jax.ShapeDtypeStruct((batch * seq * dim,), jnp.float32),
        mesh=mesh,
        scratch_types=[
            pltpu.VMEM((2, batch, _CH), jnp.float32),
            pltpu.VMEM((2, _CH), jnp.float32),
            pltpu.SemaphoreType.DMA,
            pltpu.SemaphoreType.DMA,
            pltpu.SemaphoreType.DMA,
            pltpu.SemaphoreType.DMA,
            pltpu.SemaphoreType.DMA,
            pltpu.SemaphoreType.DMA,
        ],
    )
    def run(x_hbm, pos_hbm, out_hbm, xbuf, pbuf, pin0, pin1, xin0, xin1, xout0, xout1):
        wid = lax.axis_index("s") * _NC + lax.axis_index("c")
        seg_base = wid * seg
        pins = (pin0, pin1)
        xins = (xin0, xin1)
        xouts = (xout0, xout1)

        def fire_loads(g, ss):
            pbase = seg_base + g * _CH
            pltpu.async_copy(pos_hbm.at[pl.ds(pbase, _CH)], pbuf.at[ss], pins[ss])
            for b in range(batch):
                pltpu.async_copy(
                    x_hbm.at[pl.ds(b * total + pbase, _CH)], xbuf.at[ss, b], xins[ss]
                )

        def wait_loads(ss):
            pltpu.make_async_copy(
                pos_hbm.at[pl.ds(0, _CH)], pbuf.at[ss], pins[ss]
            ).wait()
            for b in range(batch):
                pltpu.make_async_copy(
                    x_hbm.at[pl.ds(0, _CH)], xbuf.at[ss, b], xins[ss]
                ).wait()

        def compute(ss):
            @plsc.parallel_loop(0, _CH, step=_LANES, unroll=8)
            def _(o):
                pv = pbuf[ss, pl.ds(o, _LANES)]
                for b in range(batch):
                    xbuf[ss, b, pl.ds(o, _LANES)] = (
                        xbuf[ss, b, pl.ds(o, _LANES)] + pv
                    )

        def fire_outs(g, ss):
            pbase = seg_base + g * _CH
            for b in range(batch):
                pltpu.async_copy(
                    xbuf.at[ss, b], out_hbm.at[pl.ds(b * total + pbase, _CH)], xouts[ss]
                )

        def drain_outs(ss):
            for b in range(batch):
                pltpu.make_async_copy(
                    xbuf.at[ss, b], out_hbm.at[pl.ds(0, _CH)], xouts[ss]
                ).wait()

        fire_loads(0, 0)
        fire_loads(1, 1)
        wait_loads(0)
        compute(0)
        fire_outs(0, 0)

        def h_body(h, carry):
            for gg in range(2):
                g = 2 * h + 1 + gg
                ss = 1 - gg
                ss2 = gg
                drain_outs(ss2)
                fire_loads(g + 1, ss2)
                wait_loads(ss)
                compute(ss)
                fire_outs(g, ss)
            return carry

        lax.fori_loop(0, (ngrp - 2) // 2, h_body, 0)

        wait_loads((ngrp - 1) % 2)
        compute((ngrp - 1) % 2)
        fire_outs(ngrp - 1, (ngrp - 1) % 2)
        drain_outs(0)
        drain_outs(1)

    return run


def kernel(x, pos_table):
    batch, seq, dim = x.shape
    run = _sc_posadd(batch, seq, dim)
    out = run(x.reshape(-1), pos_table[:seq].reshape(-1))
    return out.reshape(batch, seq, dim)


# speedup vs baseline: 7.9268x; 4.5669x over previous
import jax
import jax.numpy as jnp
from jax.experimental import pallas as pl
from jax.experimental.pallas import tpu as pltpu

_R = 1024
_NBUF = 4
_LOOK = 2


def _make_body(batch, seq, dim):
    s_ch = seq // _R
    steps = [(s, b) for s in range(s_ch) for b in range(batch)]
    T = len(steps)

    def body(x_hbm, pos_hbm, o_hbm, xb, pb, insem, psem, outsem):
        def fire_xin(t):
            s, b = steps[t]
            pltpu.make_async_copy(
                x_hbm.at[b, pl.ds(s * _R, _R), :], xb.at[t % _NBUF],
                insem.at[t % _NBUF],
            ).start()

        def wait_xin(t):
            pltpu.make_async_copy(
                x_hbm.at[0, pl.ds(0, _R), :], xb.at[t % _NBUF],
                insem.at[t % _NBUF],
            ).wait()

        def fire_pin(s):
            pltpu.make_async_copy(
                pos_hbm.at[pl.ds(s * _R, _R), :], pb.at[s % 2], psem.at[s % 2]
            ).start()

        def wait_pin(s):
            pltpu.make_async_copy(
                pos_hbm.at[pl.ds(0, _R), :], pb.at[s % 2], psem.at[s % 2]
            ).wait()

        def fire_out(t):
            s, b = steps[t]
            pltpu.make_async_copy(
                xb.at[t % _NBUF], o_hbm.at[b, pl.ds(s * _R, _R), :],
                outsem.at[t % _NBUF],
            ).start()

        def wait_out(t):
            pltpu.make_async_copy(
                xb.at[t % _NBUF], o_hbm.at[0, pl.ds(0, _R), :],
                outsem.at[t % _NBUF],
            ).wait()

        fire_pin(0)
        fire_pin(1)
        for t in range(_LOOK):
            fire_xin(t)

        last_waited = -1
        for t in range(T):
            if t + _LOOK < T:
                if t - _NBUF + _LOOK >= 0:
                    wait_out(t - _NBUF + _LOOK)
                    last_waited = t - _NBUF + _LOOK
                fire_xin(t + _LOOK)
            s, b = steps[t]
            if b == 0:
                wait_pin(s)
            wait_xin(t)
            xb[t % _NBUF] = xb[t % _NBUF] + pb[s % 2]
            fire_out(t)
            if b == batch - 1 and s + 2 < s_ch:
                fire_pin(s + 2)

        for t in range(last_waited + 1, T):
            wait_out(t)

    return body


def kernel(x, pos_table):
    batch, seq, dim = x.shape
    body = _make_body(batch, seq, dim)
    return pl.pallas_call(
        body,
        in_specs=[
            pl.BlockSpec(memory_space=pl.ANY),
            pl.BlockSpec(memory_space=pl.ANY),
        ],
        out_specs=pl.BlockSpec(memory_space=pl.ANY),
        out_shape=jax.ShapeDtypeStruct((batch, seq, dim), x.dtype),
        scratch_shapes=[
            pltpu.VMEM((_NBUF, _R, dim), jnp.float32),
            pltpu.VMEM((2, _R, dim), jnp.float32),
            pltpu.SemaphoreType.DMA((_NBUF,)),
            pltpu.SemaphoreType.DMA((2,)),
            pltpu.SemaphoreType.DMA((_NBUF,)),
        ],
    )(x, pos_table)


# speedup vs baseline: 7.9536x; 1.0034x over previous
import jax
import jax.numpy as jnp
from jax.experimental import pallas as pl
from jax.experimental.pallas import tpu as pltpu

_R = 2048
_NBUF = 4
_LOOK = 2


def _make_body(batch, seq, dim):
    s_ch = seq // _R
    steps = [(s, b) for s in range(s_ch) for b in range(batch)]
    T = len(steps)

    def body(x_hbm, pos_hbm, o_hbm, xb, pb, insem, psem, outsem):
        def fire_xin(t):
            s, b = steps[t]
            pltpu.make_async_copy(
                x_hbm.at[b, pl.ds(s * _R, _R), :], xb.at[t % _NBUF],
                insem.at[t % _NBUF],
            ).start()

        def wait_xin(t):
            pltpu.make_async_copy(
                x_hbm.at[0, pl.ds(0, _R), :], xb.at[t % _NBUF],
                insem.at[t % _NBUF],
            ).wait()

        def fire_pin(s):
            pltpu.make_async_copy(
                pos_hbm.at[pl.ds(s * _R, _R), :], pb.at[s % 2], psem.at[s % 2]
            ).start()

        def wait_pin(s):
            pltpu.make_async_copy(
                pos_hbm.at[pl.ds(0, _R), :], pb.at[s % 2], psem.at[s % 2]
            ).wait()

        def fire_out(t):
            s, b = steps[t]
            pltpu.make_async_copy(
                xb.at[t % _NBUF], o_hbm.at[b, pl.ds(s * _R, _R), :],
                outsem.at[t % _NBUF],
            ).start()

        def wait_out(t):
            pltpu.make_async_copy(
                xb.at[t % _NBUF], o_hbm.at[0, pl.ds(0, _R), :],
                outsem.at[t % _NBUF],
            ).wait()

        fire_pin(0)
        fire_pin(1)
        for t in range(_LOOK):
            fire_xin(t)

        last_waited = -1
        for t in range(T):
            if t + _LOOK < T:
                if t - _NBUF + _LOOK >= 0:
                    wait_out(t - _NBUF + _LOOK)
                    last_waited = t - _NBUF + _LOOK
                fire_xin(t + _LOOK)
            s, b = steps[t]
            if b == 0:
                wait_pin(s)
            wait_xin(t)
            xb[t % _NBUF] = xb[t % _NBUF] + pb[s % 2]
            fire_out(t)
            if b == batch - 1 and s + 2 < s_ch:
                fire_pin(s + 2)

        for t in range(last_waited + 1, T):
            wait_out(t)

    return body


def kernel(x, pos_table):
    batch, seq, dim = x.shape
    body = _make_body(batch, seq, dim)
    return pl.pallas_call(
        body,
        in_specs=[
            pl.BlockSpec(memory_space=pl.ANY),
            pl.BlockSpec(memory_space=pl.ANY),
        ],
        out_specs=pl.BlockSpec(memory_space=pl.ANY),
        out_shape=jax.ShapeDtypeStruct((batch, seq, dim), x.dtype),
        scratch_shapes=[
            pltpu.VMEM((_NBUF, _R, dim), jnp.float32),
            pltpu.VMEM((2, _R, dim), jnp.float32),
            pltpu.SemaphoreType.DMA((_NBUF,)),
            pltpu.SemaphoreType.DMA((2,)),
            pltpu.SemaphoreType.DMA((_NBUF,)),
        ],
    )(x, pos_table)
